# baseline (speedup 1.0000x reference)
import jax
import jax.numpy as jnp
from jax.experimental import pallas as pl

_B = 8
_N = 20000
_NP = 20480
_NR = 160
_BL = 128
_TOPB = 16
_SL = _TOPB * _BL
_K = 6000
_OUT = 300
_OUTP = 384
_IOU_T = 0.7
_ONE_BITS = 1065353217


def _nms_kernel(s_ref, d_ref, a_ref, o_ref):
    s = s_ref[...]

    u = jax.lax.broadcasted_iota(jnp.int32, (4 * _BL, 4 * _BL), 0)
    v = jax.lax.broadcasted_iota(jnp.int32, (4 * _BL, 4 * _BL), 1)
    dmx = (u == 4 * (v % _BL) + v // _BL).astype(jnp.float32)

    def _exact_permute(x2d):
        hi_mask = jnp.int32(-65536)
        xi = jax.lax.bitcast_convert_type(x2d, jnp.int32)
        c1 = jax.lax.bitcast_convert_type(xi & hi_mask, jnp.float32)
        r1 = x2d - c1
        r1i = jax.lax.bitcast_convert_type(r1, jnp.int32)
        c2 = jax.lax.bitcast_convert_type(r1i & hi_mask, jnp.float32)
        c3 = r1 - c2
        out = jnp.dot(c1, dmx, preferred_element_type=jnp.float32)
        out = out + jnp.dot(c2, dmx, preferred_element_type=jnp.float32)
        return out + jnp.dot(c3, dmx, preferred_element_type=jnp.float32)

    d2 = d_ref[...].reshape(_B * _NR, 4 * _BL)
    planes = _exact_permute(d2).reshape(_B, _NR, 4 * _BL)
    dy = planes[:, :, 0:_BL] * 0.1
    dx = planes[:, :, _BL:2 * _BL] * 0.1
    dh = planes[:, :, 2 * _BL:3 * _BL] * 0.2
    dw = planes[:, :, 3 * _BL:4 * _BL] * 0.2

    aplanes = _exact_permute(a_ref[...])
    ay1 = aplanes[:, 0:_BL][None]
    ax1 = aplanes[:, _BL:2 * _BL][None]
    ay2 = aplanes[:, 2 * _BL:3 * _BL][None]
    ax2 = aplanes[:, 3 * _BL:4 * _BL][None]

    w = ax2 - ax1
    h = ay2 - ay1
    cx = ax1 + 0.5 * w
    cy = ay1 + 0.5 * h
    bw = jnp.exp(dw) * w
    bh = jnp.exp(dh) * h
    bcx = dx * w + cx
    bcy = dy * h + cy
    y1b = bcy - 0.5 * bh
    x1b = bcx - 0.5 * bw
    y2b = y1b + bh
    x2b = x1b + bw
    y1b = jnp.clip(y1b, 0.0, 1.0)
    x1b = jnp.clip(x1b, 0.0, 1.0)
    y2b = jnp.clip(y2b, 0.0, 1.0)
    x2b = jnp.clip(x2b, 0.0, 1.0)
    areab = (y2b - y1b) * (x2b - x1b)

    idx3 = (jax.lax.broadcasted_iota(jnp.int32, (_B, _NR, _BL), 1) * _BL
            + jax.lax.broadcasted_iota(jnp.int32, (_B, _NR, _BL), 2))
    riota = jax.lax.broadcasted_iota(jnp.int32, (_B, _NR, _BL), 1)
    lane = jax.lax.broadcasted_iota(jnp.int32, (_B, _BL), 1)

    m_blk = s.reshape(_B, _NR, _BL)
    ss, sy1s, sx1s, sy2s, sx2s, sidxs = [], [], [], [], [], []
    for _k in range(_TOPB):
        bmax = jnp.max(m_blk, axis=1, keepdims=True)
        pos = jnp.min(jnp.where(m_blk == bmax, riota, _NR), axis=1,
                      keepdims=True)
        oh = riota == pos
        ohf = oh.astype(jnp.float32)
        ss.append(bmax.reshape(_B, _BL))
        sy1s.append(jnp.sum(y1b * ohf, axis=1))
        sx1s.append(jnp.sum(x1b * ohf, axis=1))
        sy2s.append(jnp.sum(y2b * ohf, axis=1))
        sx2s.append(jnp.sum(x2b * ohf, axis=1))
        sidxs.append(pos.reshape(_B, _BL) * _BL + lane)
        m_blk = jnp.where(oh, -1.0, m_blk)

    gall = jnp.max(jnp.max(m_blk, axis=1), axis=1, keepdims=True)
    sl_s = jnp.concatenate(ss, axis=1)
    sl_y1 = jnp.concatenate(sy1s, axis=1)
    sl_x1 = jnp.concatenate(sx1s, axis=1)
    sl_y2 = jnp.concatenate(sy2s, axis=1)
    sl_x2 = jnp.concatenate(sx2s, axis=1)
    sl_idx = jnp.concatenate(sidxs, axis=1)
    sl_area = (sl_y2 - sl_y1) * (sl_x2 - sl_x1)

    oiota = jax.lax.broadcasted_iota(jnp.int32, (_B, _OUTP), 1)

    def _slstep(t, carry):
        m, o0, o1, o2, o3, flag = carry
        mv = jnp.max(m, axis=1, keepdims=True)
        ok = mv >= 0.0
        flag = jnp.maximum(flag, (mv <= gall).astype(jnp.float32))
        pos = jnp.min(jnp.where(m == mv, sl_idx, _NP), axis=1, keepdims=True)
        oh = (sl_idx == pos) & ok
        ohf = oh.astype(jnp.float32)
        sy1 = jnp.sum(sl_y1 * ohf, axis=1, keepdims=True)
        sx1 = jnp.sum(sl_x1 * ohf, axis=1, keepdims=True)
        sy2 = jnp.sum(sl_y2 * ohf, axis=1, keepdims=True)
        sx2 = jnp.sum(sl_x2 * ohf, axis=1, keepdims=True)
        sarea = (sy2 - sy1) * (sx2 - sx1)
        yy1 = jnp.maximum(sy1, sl_y1)
        xx1 = jnp.maximum(sx1, sl_x1)
        yy2 = jnp.minimum(sy2, sl_y2)
        xx2 = jnp.minimum(sx2, sl_x2)
        inter = jnp.maximum(yy2 - yy1, 0.0) * jnp.maximum(xx2 - xx1, 0.0)
        iou = inter / (sarea + sl_area - inter + 1e-8)
        supp = (iou > _IOU_T) | oh
        m = jnp.where(supp & ok, -1.0, m)
        tm = (oiota == t).astype(jnp.float32)
        o0 = o0 + sy1 * tm
        o1 = o1 + sx1 * tm
        o2 = o2 + sy2 * tm
        o3 = o3 + sx2 * tm
        return m, o0, o1, o2, o3, flag

    z = jnp.zeros((_B, _OUTP), jnp.float32)
    flag0 = jnp.zeros((_B, 1), jnp.float32)
    _, s0, s1, s2, s3, flag = jax.lax.fori_loop(
        0, _OUT, _slstep, (sl_s, z, z, z, z, flag0))

    fb = jnp.max(flag) > 0.0
    s3d = s.reshape(_B, _NR, _BL)
    bits = jax.lax.bitcast_convert_type(s3d, jnp.int32)

    def _cnt3(x):
        return jnp.sum(jnp.sum(x.astype(jnp.int32), axis=2, keepdims=True),
                       axis=1, keepdims=True)

    def _tstep(_, lohi):
        lo, hi = lohi
        mid = (lo + hi) // 2
        ge = _cnt3(bits >= mid) >= _K
        return jnp.where(ge, mid, lo), jnp.where(ge, hi, mid)

    lo0 = jnp.zeros((_B, 1, 1), jnp.int32)
    hi0 = jnp.full((_B, 1, 1), _ONE_BITS, jnp.int32)
    vstar, _ = jax.lax.fori_loop(0, jnp.where(fb, 31, 0), _tstep, (lo0, hi0))

    cnt_gt = _cnt3(bits > vstar)
    quota = _K - cnt_gt
    tie = bits == vstar

    def _istep(_, lohi):
        lo, hi = lohi
        mid = (lo + hi) // 2
        ge = _cnt3(tie & (idx3 <= mid)) >= quota
        return jnp.where(ge, lo, mid + 1), jnp.where(ge, mid, hi)

    lo0 = jnp.zeros((_B, 1, 1), jnp.int32)
    hi0 = jnp.full((_B, 1, 1), _NP - 1, jnp.int32)
    istar, _ = jax.lax.fori_loop(0, jnp.where(fb, 16, 0), _istep, (lo0, hi0))

    eligible = (bits > vstar) | (tie & (idx3 <= istar))
    m0 = jnp.where(eligible, s3d, -1.0)

    def _max3(x):
        return jnp.max(jnp.max(x, axis=2, keepdims=True), axis=1,
                       keepdims=True)

    def _min3(x):
        return jnp.min(jnp.min(x, axis=2, keepdims=True), axis=1,
                       keepdims=True)

    def _sum3(x):
        return jnp.sum(jnp.sum(x, axis=2, keepdims=True), axis=1,
                       keepdims=True)

    def _fullstep(t, carry):
        m, o0, o1, o2, o3 = carry
        mv = _max3(m)
        ok = mv >= 0.0
        pos = _min3(jnp.where(m == mv, idx3, _NP))
        oh = (idx3 == pos) & ok
        ohf = oh.astype(jnp.float32)
        sy1 = _sum3(y1b * ohf)
        sx1 = _sum3(x1b * ohf)
        sy2 = _sum3(y2b * ohf)
        sx2 = _sum3(x2b * ohf)
        sarea = (sy2 - sy1) * (sx2 - sx1)
        yy1 = jnp.maximum(sy1, y1b)
        xx1 = jnp.maximum(sx1, x1b)
        yy2 = jnp.minimum(sy2, y2b)
        xx2 = jnp.minimum(sx2, x2b)
        inter = jnp.maximum(yy2 - yy1, 0.0) * jnp.maximum(xx2 - xx1, 0.0)
        iou = inter / (sarea + areab - inter + 1e-8)
        supp = (iou > _IOU_T) | oh
        m = jnp.where(supp & ok, -1.0, m)
        tm = (oiota == t).astype(jnp.float32)
        o0 = o0 + sy1.reshape(_B, 1) * tm
        o1 = o1 + sx1.reshape(_B, 1) * tm
        o2 = o2 + sy2.reshape(_B, 1) * tm
        o3 = o3 + sx2.reshape(_B, 1) * tm
        return m, o0, o1, o2, o3

    _, f0, f1, f2, f3 = jax.lax.fori_loop(
        0, jnp.where(fb, _OUT, 0), _fullstep, (m0, z, z, z, z))
    use_fb = flag > 0.0
    o_ref[0] = jnp.where(use_fb, f0, s0)
    o_ref[1] = jnp.where(use_fb, f1, s1)
    o_ref[2] = jnp.where(use_fb, f2, s2)
    o_ref[3] = jnp.where(use_fb, f3, s3)


def kernel(rpn_bbox_deltas, rpn_labels, anchors):
    scores = rpn_labels.reshape(_B, _N)
    padn = _NP - _N
    d_flat = jnp.pad(rpn_bbox_deltas.reshape(_B, _N * 4),
                     ((0, 0), (0, padn * 4)))
    d4 = d_flat.reshape(_B, _NR, 4 * _BL)
    a_flat = jnp.pad(anchors.reshape(_N * 4), (0, padn * 4))
    a4 = a_flat.reshape(_NR, 4 * _BL)
    s_p = jnp.pad(scores, ((0, 0), (0, padn)), constant_values=-1.0)
    out = pl.pallas_call(
        _nms_kernel,
        out_shape=jax.ShapeDtypeStruct((4, _B, _OUTP), jnp.float32),
    )(s_p, d4, a4)
    roi = jnp.transpose(out, (1, 2, 0))[:, :_OUT, :]
    return jax.lax.stop_gradient(roi)


# speedup vs baseline: 1.0130x; 1.0130x over previous
import jax
import jax.numpy as jnp
from jax.experimental import pallas as pl

_B = 8
_N = 20000
_NP = 20480
_NR = 160
_BL = 128
_TOPB = 16
_SL = _TOPB * _BL
_K = 6000
_OUT = 300
_OUTP = 384
_IOU_T = 0.7
_ONE_BITS = 1065353217


def _nms_kernel(s_ref, d_ref, a_ref, o_ref):
    s = s_ref[...]

    u = jax.lax.broadcasted_iota(jnp.int32, (4 * _BL, 4 * _BL), 0)
    v = jax.lax.broadcasted_iota(jnp.int32, (4 * _BL, 4 * _BL), 1)
    dmx = (u == 4 * (v % _BL) + v // _BL).astype(jnp.float32)

    def _exact_permute(x2d):
        hi_mask = jnp.int32(-65536)
        xi = jax.lax.bitcast_convert_type(x2d, jnp.int32)
        c1 = jax.lax.bitcast_convert_type(xi & hi_mask, jnp.float32)
        r1 = x2d - c1
        r1i = jax.lax.bitcast_convert_type(r1, jnp.int32)
        c2 = jax.lax.bitcast_convert_type(r1i & hi_mask, jnp.float32)
        c3 = r1 - c2
        out = jnp.dot(c1, dmx, preferred_element_type=jnp.float32)
        out = out + jnp.dot(c2, dmx, preferred_element_type=jnp.float32)
        return out + jnp.dot(c3, dmx, preferred_element_type=jnp.float32)

    d2 = d_ref[...].reshape(_B * _NR, 4 * _BL)
    planes = _exact_permute(d2).reshape(_B, _NR, 4 * _BL)
    dy = planes[:, :, 0:_BL] * 0.1
    dx = planes[:, :, _BL:2 * _BL] * 0.1
    dh = planes[:, :, 2 * _BL:3 * _BL] * 0.2
    dw = planes[:, :, 3 * _BL:4 * _BL] * 0.2

    aplanes = _exact_permute(a_ref[...])
    ay1 = aplanes[:, 0:_BL][None]
    ax1 = aplanes[:, _BL:2 * _BL][None]
    ay2 = aplanes[:, 2 * _BL:3 * _BL][None]
    ax2 = aplanes[:, 3 * _BL:4 * _BL][None]

    w = ax2 - ax1
    h = ay2 - ay1
    cx = ax1 + 0.5 * w
    cy = ay1 + 0.5 * h
    bw = jnp.exp(dw) * w
    bh = jnp.exp(dh) * h
    bcx = dx * w + cx
    bcy = dy * h + cy
    y1b = bcy - 0.5 * bh
    x1b = bcx - 0.5 * bw
    y2b = y1b + bh
    x2b = x1b + bw
    y1b = jnp.clip(y1b, 0.0, 1.0)
    x1b = jnp.clip(x1b, 0.0, 1.0)
    y2b = jnp.clip(y2b, 0.0, 1.0)
    x2b = jnp.clip(x2b, 0.0, 1.0)
    areab = (y2b - y1b) * (x2b - x1b)

    idx3 = (jax.lax.broadcasted_iota(jnp.int32, (_B, _NR, _BL), 1) * _BL
            + jax.lax.broadcasted_iota(jnp.int32, (_B, _NR, _BL), 2))
    riota = jax.lax.broadcasted_iota(jnp.int32, (_B, _NR, _BL), 1)
    lane = jax.lax.broadcasted_iota(jnp.int32, (_B, _BL), 1)

    m_blk = s.reshape(_B, _NR, _BL)
    ss, sy1s, sx1s, sy2s, sx2s, sidxs = [], [], [], [], [], []
    for _k in range(_TOPB):
        bmax = jnp.max(m_blk, axis=1, keepdims=True)
        pos = jnp.min(jnp.where(m_blk == bmax, riota, _NR), axis=1,
                      keepdims=True)
        oh = riota == pos
        ohf = oh.astype(jnp.float32)
        ss.append(bmax.reshape(_B, _BL))
        sy1s.append(jnp.sum(y1b * ohf, axis=1))
        sx1s.append(jnp.sum(x1b * ohf, axis=1))
        sy2s.append(jnp.sum(y2b * ohf, axis=1))
        sx2s.append(jnp.sum(x2b * ohf, axis=1))
        sidxs.append(pos.reshape(_B, _BL) * _BL + lane)
        m_blk = jnp.where(oh, -1.0, m_blk)

    gall = jnp.max(jnp.max(m_blk, axis=1), axis=1, keepdims=True)
    sl_s = jnp.concatenate(ss, axis=1)
    sl_y1 = jnp.concatenate(sy1s, axis=1)
    sl_x1 = jnp.concatenate(sx1s, axis=1)
    sl_y2 = jnp.concatenate(sy2s, axis=1)
    sl_x2 = jnp.concatenate(sx2s, axis=1)
    sl_idx = jnp.concatenate(sidxs, axis=1)
    sl_area = (sl_y2 - sl_y1) * (sl_x2 - sl_x1)

    oiota = jax.lax.broadcasted_iota(jnp.int32, (_B, _OUTP), 1)

    def _slstep(t, carry):
        m, o0, o1, o2, o3, flag = carry
        mv = jnp.max(m, axis=1, keepdims=True)
        ok = mv >= 0.0
        flag = jnp.maximum(flag, (mv <= gall).astype(jnp.float32))
        pos = jnp.min(jnp.where(m == mv, sl_idx, _NP), axis=1, keepdims=True)
        oh = (sl_idx == pos) & ok
        ohf = oh.astype(jnp.float32)
        sy1 = jnp.sum(sl_y1 * ohf, axis=1, keepdims=True)
        sx1 = jnp.sum(sl_x1 * ohf, axis=1, keepdims=True)
        sy2 = jnp.sum(sl_y2 * ohf, axis=1, keepdims=True)
        sx2 = jnp.sum(sl_x2 * ohf, axis=1, keepdims=True)
        sarea = (sy2 - sy1) * (sx2 - sx1)
        yy1 = jnp.maximum(sy1, sl_y1)
        xx1 = jnp.maximum(sx1, sl_x1)
        yy2 = jnp.minimum(sy2, sl_y2)
        xx2 = jnp.minimum(sx2, sl_x2)
        inter = jnp.maximum(yy2 - yy1, 0.0) * jnp.maximum(xx2 - xx1, 0.0)
        iou = inter / (sarea + sl_area - inter + 1e-8)
        supp = (iou > _IOU_T) | oh
        m = jnp.where(supp & ok, -1.0, m)
        tm = (oiota == t).astype(jnp.float32)
        o0 = o0 + sy1 * tm
        o1 = o1 + sx1 * tm
        o2 = o2 + sy2 * tm
        o3 = o3 + sx2 * tm
        return m, o0, o1, o2, o3, flag

    z = jnp.zeros((_B, _OUTP), jnp.float32)
    flag0 = jnp.zeros((_B, 1), jnp.float32)
    _, s0, s1, s2, s3, flag = jax.lax.fori_loop(
        0, _OUT, _slstep, (sl_s, z, z, z, z, flag0), unroll=4)

    fb = jnp.max(flag) > 0.0
    s3d = s.reshape(_B, _NR, _BL)
    bits = jax.lax.bitcast_convert_type(s3d, jnp.int32)

    def _cnt3(x):
        return jnp.sum(jnp.sum(x.astype(jnp.int32), axis=2, keepdims=True),
                       axis=1, keepdims=True)

    def _tstep(_, lohi):
        lo, hi = lohi
        mid = (lo + hi) // 2
        ge = _cnt3(bits >= mid) >= _K
        return jnp.where(ge, mid, lo), jnp.where(ge, hi, mid)

    lo0 = jnp.zeros((_B, 1, 1), jnp.int32)
    hi0 = jnp.full((_B, 1, 1), _ONE_BITS, jnp.int32)
    vstar, _ = jax.lax.fori_loop(0, jnp.where(fb, 31, 0), _tstep, (lo0, hi0))

    cnt_gt = _cnt3(bits > vstar)
    quota = _K - cnt_gt
    tie = bits == vstar

    def _istep(_, lohi):
        lo, hi = lohi
        mid = (lo + hi) // 2
        ge = _cnt3(tie & (idx3 <= mid)) >= quota
        return jnp.where(ge, lo, mid + 1), jnp.where(ge, mid, hi)

    lo0 = jnp.zeros((_B, 1, 1), jnp.int32)
    hi0 = jnp.full((_B, 1, 1), _NP - 1, jnp.int32)
    istar, _ = jax.lax.fori_loop(0, jnp.where(fb, 16, 0), _istep, (lo0, hi0))

    eligible = (bits > vstar) | (tie & (idx3 <= istar))
    m0 = jnp.where(eligible, s3d, -1.0)

    def _max3(x):
        return jnp.max(jnp.max(x, axis=2, keepdims=True), axis=1,
                       keepdims=True)

    def _min3(x):
        return jnp.min(jnp.min(x, axis=2, keepdims=True), axis=1,
                       keepdims=True)

    def _sum3(x):
        return jnp.sum(jnp.sum(x, axis=2, keepdims=True), axis=1,
                       keepdims=True)

    def _fullstep(t, carry):
        m, o0, o1, o2, o3 = carry
        mv = _max3(m)
        ok = mv >= 0.0
        pos = _min3(jnp.where(m == mv, idx3, _NP))
        oh = (idx3 == pos) & ok
        ohf = oh.astype(jnp.float32)
        sy1 = _sum3(y1b * ohf)
        sx1 = _sum3(x1b * ohf)
        sy2 = _sum3(y2b * ohf)
        sx2 = _sum3(x2b * ohf)
        sarea = (sy2 - sy1) * (sx2 - sx1)
        yy1 = jnp.maximum(sy1, y1b)
        xx1 = jnp.maximum(sx1, x1b)
        yy2 = jnp.minimum(sy2, y2b)
        xx2 = jnp.minimum(sx2, x2b)
        inter = jnp.maximum(yy2 - yy1, 0.0) * jnp.maximum(xx2 - xx1, 0.0)
        iou = inter / (sarea + areab - inter + 1e-8)
        supp = (iou > _IOU_T) | oh
        m = jnp.where(supp & ok, -1.0, m)
        tm = (oiota == t).astype(jnp.float32)
        o0 = o0 + sy1.reshape(_B, 1) * tm
        o1 = o1 + sx1.reshape(_B, 1) * tm
        o2 = o2 + sy2.reshape(_B, 1) * tm
        o3 = o3 + sx2.reshape(_B, 1) * tm
        return m, o0, o1, o2, o3

    _, f0, f1, f2, f3 = jax.lax.fori_loop(
        0, jnp.where(fb, _OUT, 0), _fullstep, (m0, z, z, z, z))
    use_fb = flag > 0.0
    o_ref[0] = jnp.where(use_fb, f0, s0)
    o_ref[1] = jnp.where(use_fb, f1, s1)
    o_ref[2] = jnp.where(use_fb, f2, s2)
    o_ref[3] = jnp.where(use_fb, f3, s3)


def kernel(rpn_bbox_deltas, rpn_labels, anchors):
    scores = rpn_labels.reshape(_B, _N)
    padn = _NP - _N
    d_flat = jnp.pad(rpn_bbox_deltas.reshape(_B, _N * 4),
                     ((0, 0), (0, padn * 4)))
    d4 = d_flat.reshape(_B, _NR, 4 * _BL)
    a_flat = jnp.pad(anchors.reshape(_N * 4), (0, padn * 4))
    a4 = a_flat.reshape(_NR, 4 * _BL)
    s_p = jnp.pad(scores, ((0, 0), (0, padn)), constant_values=-1.0)
    out = pl.pallas_call(
        _nms_kernel,
        out_shape=jax.ShapeDtypeStruct((4, _B, _OUTP), jnp.float32),
    )(s_p, d4, a4)
    roi = jnp.transpose(out, (1, 2, 0))[:, :_OUT, :]
    return jax.lax.stop_gradient(roi)
